# baseline (speedup 1.0000x reference)
import jax
import jax.numpy as jnp
from jax import lax
from jax.experimental import pallas as pl
from jax.experimental.pallas import tpu as pltpu
from jax.experimental.pallas import tpu_sc as plsc

N = 10000
E = 320000
D_IN = 128
D_HID = 64
D_OUT = 128

NC = 2
NS = 16
NW = NC * NS
CHUNK = 128
NBUF = 4
K = 80
E_PAD = NW * K * CHUNK
EW = K * CHUNK
DUMMY = N
RPT = 640
N_ACC = NS * RPT


def _make_sc_seg(with_count: bool):
    mesh = plsc.VectorSubcoreMesh(core_axis_name="c", subcore_axis_name="s")
    acc_t = jax.ShapeDtypeStruct((NC, N_ACC, D_HID), jnp.float32)
    if with_count:
        out_type = (acc_t, jax.ShapeDtypeStruct((NW, EW), jnp.float32))
    else:
        out_type = acc_t

    def body(table_hbm, src_hbm, dst_hbm, zrow_hbm, zh_hbm, *rest):
        if with_count:
            out_hbm, cnt_hbm = rest[0], rest[1]
            scr = rest[2:]
        else:
            out_hbm = rest[0]
            scr = rest[1:]
        srcv, dstv, hist1 = scr[:3]
        rows = scr[3:3 + NBUF]
        gsems = scr[3 + NBUF:3 + 2 * NBUF]
        acc_sh = scr[-1]
        cid = lax.axis_index("c")
        sid = lax.axis_index("s")
        wid = sid * NC + cid

        pltpu.sync_copy(zrow_hbm, acc_sh.at[pl.ds(sid * RPT, RPT)])
        if with_count:
            pltpu.sync_copy(zh_hbm, hist1)
        pltpu.sync_copy(src_hbm.at[wid], srcv)
        pltpu.sync_copy(dst_hbm.at[wid], dstv)
        plsc.subcore_barrier()

        def outer(i, carry):
            base = i * NBUF
            gathers = [
                pltpu.async_copy(
                    table_hbm.at[srcv.at[base + b]], rows[b], gsems[b])
                for b in range(NBUF)
            ]
            for b in range(NBUF):
                gathers[b].wait()
                pltpu.sync_copy(rows[b], acc_sh.at[dstv.at[base + b]],
                                add=True)
            return carry

        lax.fori_loop(0, K // NBUF, outer, 0)

        if with_count:
            ones16 = jnp.full((16,), 1.0, jnp.float32)

            def hstep(j, carry):
                for l in range(CHUNK // 16):
                    idx = dstv[j, pl.ds(l * 16, 16)]
                    plsc.addupdate_scatter(hist1, [idx], ones16)
                return carry

            lax.fori_loop(0, K, hstep, 0)
            pltpu.sync_copy(hist1, cnt_hbm.at[wid])

        plsc.subcore_barrier()

        pltpu.sync_copy(acc_sh.at[pl.ds(sid * RPT, RPT)],
                        out_hbm.at[cid, pl.ds(sid * RPT, RPT)])

    return pl.kernel(
        body,
        mesh=mesh,
        out_type=out_type,
        compiler_params=pltpu.CompilerParams(use_tc_tiling_on_sc=False,
                                             needs_layout_passes=False),
        scratch_types=(
            [pltpu.VMEM((K, CHUNK), jnp.int32),
             pltpu.VMEM((K, CHUNK), jnp.int32),
             pltpu.VMEM((EW,), jnp.float32)]
            + [pltpu.VMEM((CHUNK, D_HID), jnp.float32) for _ in range(NBUF)]
            + [pltpu.SemaphoreType.DMA for _ in range(NBUF)]
            + [pltpu.VMEM_SHARED((N_ACC, D_HID), jnp.float32)]
        ),
    )


_sc_seg_l1 = _make_sc_seg(True)
_sc_seg_l2 = _make_sc_seg(False)


def _dense_a_body(x_ref, wn_ref, wr_ref, b_ref, xa_ref, xr_ref):
    x = x_ref[...]
    xa_ref[...] = jnp.dot(x, wn_ref[...], preferred_element_type=jnp.float32)
    xr_ref[...] = (jnp.dot(x, wr_ref[...], preferred_element_type=jnp.float32)
                   + b_ref[...])


def _dense_a(x, wn, wr, b):
    return pl.pallas_call(
        _dense_a_body,
        out_shape=(jax.ShapeDtypeStruct((N, D_HID), jnp.float32),
                   jax.ShapeDtypeStruct((N, D_HID), jnp.float32)),
    )(x, wn, wr, b)


def _dense_b0_body(ch_ref, cs_ref):
    cs_ref[...] = jnp.sum(ch_ref[...], axis=0, keepdims=True)


def _dense_b0(cnth):
    return pl.pallas_call(
        _dense_b0_body,
        out_shape=jax.ShapeDtypeStruct((1, EW), jnp.float32),
    )(cnth)


def _dense_b_body(p0_ref, p1_ref, c_ref, xr_ref, h_ref, rinv_ref):
    cnt = jnp.maximum(c_ref[...], 1.0)
    rinv = 1.0 / cnt
    rinv_ref[...] = rinv
    h_ref[...] = jnp.maximum(
        (p0_ref[...] + p1_ref[...]) * rinv + xr_ref[...], 0.0)


def _dense_b(p0, p1, c, xr):
    return pl.pallas_call(
        _dense_b_body,
        out_shape=(jax.ShapeDtypeStruct((N, D_HID), jnp.float32),
                   jax.ShapeDtypeStruct((N, 1), jnp.float32)),
    )(p0, p1, c, xr)


def _dense_c_body(q0_ref, q1_ref, rinv_ref, h_ref, wn_ref, wr_ref, b_ref,
                  out_ref):
    mean2 = (q0_ref[...] + q1_ref[...]) * rinv_ref[...]
    z = (jnp.dot(mean2, wn_ref[...], preferred_element_type=jnp.float32)
         + jnp.dot(h_ref[...], wr_ref[...], preferred_element_type=jnp.float32)
         + b_ref[...])
    z = jnp.maximum(z, 0.0)
    z = z - jnp.max(z, axis=1, keepdims=True)
    out_ref[...] = z - jnp.log(jnp.sum(jnp.exp(z), axis=1, keepdims=True))


def _dense_c(q0, q1, rinv, h, wn, wr, b):
    return pl.pallas_call(
        _dense_c_body,
        out_shape=jax.ShapeDtypeStruct((N, D_OUT), jnp.float32),
    )(q0, q1, rinv, h, wn, wr, b)


def kernel(x, edge_index, W1_nei, W1_root, b1, W2_nei, W2_root, b2):
    src = edge_index[0].astype(jnp.int32)
    dst = edge_index[1].astype(jnp.int32)
    pad = E_PAD - E
    src_p = jnp.concatenate([src, jnp.zeros((pad,), jnp.int32)]
                            ).reshape(NW, K, CHUNK)
    dst_p = jnp.concatenate([dst, jnp.full((pad,), DUMMY, jnp.int32)]
                            ).reshape(NW, K, CHUNK)
    zrow = jnp.zeros((RPT, D_HID), jnp.float32)
    zh = jnp.zeros((EW,), jnp.float32)

    xa, xr = _dense_a(x, W1_nei, W1_root, b1.reshape(1, D_HID))
    parts, cnth = _sc_seg_l1(xa, src_p, dst_p, zrow, zh)
    csum = _dense_b0(cnth)
    c_col = csum.reshape(EW, 1)[:N]
    h, rinv = _dense_b(parts[0, :N], parts[1, :N], c_col, xr)
    parts2 = _sc_seg_l2(h, src_p, dst_p, zrow, zh)
    out = _dense_c(parts2[0, :N], parts2[1, :N], rinv, h,
                   W2_nei, W2_root, b2.reshape(1, D_OUT))
    return out


# speedup vs baseline: 1.0274x; 1.0274x over previous
import jax
import jax.numpy as jnp
from jax import lax
from jax.experimental import pallas as pl
from jax.experimental.pallas import tpu as pltpu
from jax.experimental.pallas import tpu_sc as plsc

N = 10000
E = 320000
D_IN = 128
D_HID = 64
D_OUT = 128

NC = 2
NS = 16
NW = NC * NS
CHUNK = 128
NBUF = 6
K = 78
EW = 10240
KT = (EW - K * CHUNK) // CHUNK
E_PAD = NW * EW
DUMMY = N
RPT = 640
N_ACC = NS * RPT


def _make_sc_seg(with_count: bool):
    mesh = plsc.VectorSubcoreMesh(core_axis_name="c", subcore_axis_name="s")
    acc_t = jax.ShapeDtypeStruct((NC, N_ACC, D_HID), jnp.float32)
    if with_count:
        out_type = (acc_t, jax.ShapeDtypeStruct((NC, N_ACC), jnp.float32))
    else:
        out_type = acc_t

    def body(table_hbm, src_hbm, dst_hbm, zrow_hbm, z1_hbm, ones_hbm, *rest):
        if with_count:
            out_hbm, cnt_hbm = rest[0], rest[1]
            scr = rest[2:]
        else:
            out_hbm = rest[0]
            scr = rest[1:]
        srcv, dstv, onesv = scr[:3]
        rows = scr[3:3 + NBUF]
        gsems = scr[3 + NBUF:3 + 2 * NBUF]
        acc_sh, cnt_sh = scr[-2], scr[-1]
        cid = lax.axis_index("c")
        sid = lax.axis_index("s")
        wid = sid * NC + cid

        pltpu.sync_copy(zrow_hbm, acc_sh.at[pl.ds(sid * RPT, RPT)])
        if with_count:
            pltpu.sync_copy(z1_hbm, cnt_sh.at[pl.ds(sid * RPT, RPT)])
            pltpu.sync_copy(ones_hbm, onesv)
        pltpu.sync_copy(src_hbm.at[wid], srcv)
        pltpu.sync_copy(dst_hbm.at[wid], dstv)
        plsc.subcore_barrier()

        def outer(i, carry):
            base = i * NBUF
            gathers = [
                pltpu.async_copy(
                    table_hbm.at[srcv.at[base + b]], rows[b], gsems[b])
                for b in range(NBUF)
            ]
            for b in range(NBUF):
                gathers[b].wait()
                pltpu.sync_copy(rows[b], acc_sh.at[dstv.at[base + b]],
                                add=True)
                if with_count:
                    pltpu.sync_copy(onesv, cnt_sh.at[dstv.at[base + b]],
                                    add=True)
            return carry

        lax.fori_loop(0, K // NBUF, outer, 0)

        tail = [
            pltpu.async_copy(
                table_hbm.at[srcv.at[K + t]], rows[t], gsems[t])
            for t in range(KT)
        ]
        for t in range(KT):
            tail[t].wait()
            pltpu.sync_copy(rows[t], acc_sh.at[dstv.at[K + t]], add=True)
            if with_count:
                pltpu.sync_copy(onesv, cnt_sh.at[dstv.at[K + t]], add=True)

        plsc.subcore_barrier()

        pltpu.sync_copy(acc_sh.at[pl.ds(sid * RPT, RPT)],
                        out_hbm.at[cid, pl.ds(sid * RPT, RPT)])
        if with_count:
            pltpu.sync_copy(cnt_sh.at[pl.ds(sid * RPT, RPT)],
                            cnt_hbm.at[cid, pl.ds(sid * RPT, RPT)])

    return pl.kernel(
        body,
        mesh=mesh,
        out_type=out_type,
        compiler_params=pltpu.CompilerParams(use_tc_tiling_on_sc=False),
        scratch_types=(
            [pltpu.VMEM((K + KT, CHUNK), jnp.int32),
             pltpu.VMEM((K + KT, CHUNK), jnp.int32),
             pltpu.VMEM((CHUNK,), jnp.float32)]
            + [pltpu.VMEM((CHUNK, D_HID), jnp.float32) for _ in range(NBUF)]
            + [pltpu.SemaphoreType.DMA for _ in range(NBUF)]
            + [pltpu.VMEM_SHARED((N_ACC, D_HID), jnp.float32),
               pltpu.VMEM_SHARED((N_ACC,), jnp.float32)]
        ),
    )


_sc_seg_l1 = _make_sc_seg(True)
_sc_seg_l2 = _make_sc_seg(False)


def _dense_a1_body(x_ref, wn_ref, xa_ref):
    xa_ref[...] = jnp.dot(x_ref[...], wn_ref[...],
                          preferred_element_type=jnp.float32)


def _dense_a1(x, wn):
    return pl.pallas_call(
        _dense_a1_body,
        out_shape=jax.ShapeDtypeStruct((N, D_HID), jnp.float32),
    )(x, wn)


def _dense_a2_body(x_ref, wr_ref, b_ref, xr_ref):
    xr_ref[...] = (jnp.dot(x_ref[...], wr_ref[...],
                           preferred_element_type=jnp.float32) + b_ref[...])


def _dense_a2(x, wr, b):
    return pl.pallas_call(
        _dense_a2_body,
        out_shape=jax.ShapeDtypeStruct((N, D_HID), jnp.float32),
    )(x, wr, b)


def _dense_b_body(p0_ref, p1_ref, c0_ref, c1_ref, xr_ref, h_ref, rinv_ref):
    cnt = jnp.maximum(c0_ref[...] + c1_ref[...], 1.0)
    rinv = 1.0 / cnt
    rinv_ref[...] = rinv
    h_ref[...] = jnp.maximum(
        (p0_ref[...] + p1_ref[...]) * rinv + xr_ref[...], 0.0)


def _dense_b(p0, p1, c0, c1, xr):
    return pl.pallas_call(
        _dense_b_body,
        out_shape=(jax.ShapeDtypeStruct((N, D_HID), jnp.float32),
                   jax.ShapeDtypeStruct((N, 1), jnp.float32)),
    )(p0, p1, c0, c1, xr)


def _dense_c_body(q0_ref, q1_ref, rinv_ref, h_ref, wn_ref, wr_ref, b_ref,
                  out_ref):
    mean2 = (q0_ref[...] + q1_ref[...]) * rinv_ref[...]
    z = (jnp.dot(mean2, wn_ref[...], preferred_element_type=jnp.float32)
         + jnp.dot(h_ref[...], wr_ref[...], preferred_element_type=jnp.float32)
         + b_ref[...])
    z = jnp.maximum(z, 0.0)
    z = z - jnp.max(z, axis=1, keepdims=True)
    out_ref[...] = z - jnp.log(jnp.sum(jnp.exp(z), axis=1, keepdims=True))


def _dense_c(q0, q1, rinv, h, wn, wr, b):
    return pl.pallas_call(
        _dense_c_body,
        out_shape=jax.ShapeDtypeStruct((N, D_OUT), jnp.float32),
    )(q0, q1, rinv, h, wn, wr, b)


def kernel(x, edge_index, W1_nei, W1_root, b1, W2_nei, W2_root, b2):
    src = edge_index[0].astype(jnp.int32)
    dst = edge_index[1].astype(jnp.int32)
    pad = E_PAD - E
    src_p = jnp.concatenate([src, jnp.zeros((pad,), jnp.int32)]
                            ).reshape(NW, K + KT, CHUNK)
    dst_p = jnp.concatenate([dst, jnp.full((pad,), DUMMY, jnp.int32)]
                            ).reshape(NW, K + KT, CHUNK)
    zrow = jnp.zeros((RPT, D_HID), jnp.float32)
    z1 = jnp.zeros((RPT,), jnp.float32)
    ones_c = jnp.ones((CHUNK,), jnp.float32)

    xa = _dense_a1(x, W1_nei)
    xr = _dense_a2(x, W1_root, b1.reshape(1, D_HID))
    parts, cnts = _sc_seg_l1(xa, src_p, dst_p, zrow, z1, ones_c)
    h, rinv = _dense_b(parts[0, :N], parts[1, :N],
                       cnts[0, :N, None], cnts[1, :N, None], xr)
    parts2 = _sc_seg_l2(h, src_p, dst_p, zrow, z1, ones_c)
    out = _dense_c(parts2[0, :N], parts2[1, :N], rinv, h,
                   W2_nei, W2_root, b2.reshape(1, D_OUT))
    return out
